# Initial kernel scaffold; baseline (speedup 1.0000x reference)
import jax
import jax.numpy as jnp
from jax.experimental import pallas as pl


def kernel(x, edge_index, W, b, gamma, beta):
    raise NotImplementedError("write your pallas kernel here")


# speedup vs baseline: 5.0970x; 5.0970x over previous
import functools

import jax
import jax.numpy as jnp
from jax import lax
from jax.experimental import pallas as pl
from jax.experimental.pallas import tpu as pltpu
from jax.experimental.pallas import tpu_sc as plsc

EPS = 1e-5

NC = 2
NS = 16
CHUNK = 80


def _matmul_body(x_ref, w_ref, h_ref):
    h_ref[...] = jnp.dot(x_ref[...], w_ref[...],
                         preferred_element_type=jnp.float32)


def _matmul(x, W, block_rows):
    n, d = x.shape
    return pl.pallas_call(
        _matmul_body,
        grid=(n // block_rows,),
        in_specs=[
            pl.BlockSpec((block_rows, d), lambda i: (i, 0)),
            pl.BlockSpec((d, d), lambda i: (0, 0)),
        ],
        out_specs=pl.BlockSpec((block_rows, d), lambda i: (i, 0)),
        out_shape=jax.ShapeDtypeStruct((n, d), jnp.float32),
    )(x, W)


def _make_sc_agg(n, d, e):
    per_w = e // (NC * NS)
    chunks = per_w // CHUNK
    rem = per_w - chunks * CHUNK
    assert rem == 0, "edge count must split evenly into chunks"
    nzch = n // CHUNK
    max_per_tile = (nzch + NS - 1) // NS

    mesh = plsc.VectorSubcoreMesh(core_axis_name="c", subcore_axis_name="s")

    @functools.partial(
        pl.kernel,
        mesh=mesh,
        out_type=jax.ShapeDtypeStruct((NC, n, d), jnp.float32),
        compiler_params=pltpu.CompilerParams(use_tc_tiling_on_sc=False),
        scratch_types=[
            pltpu.VMEM((CHUNK,), jnp.int32),
            pltpu.VMEM((CHUNK,), jnp.int32),
            pltpu.VMEM((CHUNK, d), jnp.float32),
            pltpu.VMEM_SHARED((n, d), jnp.float32),
            pltpu.SemaphoreType.DMA,
        ],
    )
    def sc_agg(h_hbm, src_hbm, dst_hbm, zero_hbm, out_hbm,
               sidx, didx, rows, acc, sem):
        cid = lax.axis_index("c")
        sid = lax.axis_index("s")

        def zbody(t, carry):
            c = sid + t * NS

            @pl.when(c < nzch)
            def _():
                pltpu.sync_copy(zero_hbm, acc.at[pl.ds(c * CHUNK, CHUNK)])

            return carry

        lax.fori_loop(0, max_per_tile, zbody, 0)
        plsc.subcore_barrier()

        base = cid * (e // NC) + sid * per_w

        def chunk_body(t, carry):
            off = base + t * CHUNK
            pltpu.sync_copy(src_hbm.at[pl.ds(off, CHUNK)], sidx)
            pltpu.sync_copy(dst_hbm.at[pl.ds(off, CHUNK)], didx)
            pltpu.async_copy(h_hbm.at[sidx], rows, sem).wait()
            pltpu.sync_copy(rows, acc.at[didx], add=True)
            return carry

        lax.fori_loop(0, chunks, chunk_body, 0)
        plsc.subcore_barrier()

        def wbody(t, carry):
            c = sid + t * NS

            @pl.when(c < nzch)
            def _():
                pltpu.sync_copy(acc.at[pl.ds(c * CHUNK, CHUNK)],
                                out_hbm.at[cid, pl.ds(c * CHUNK, CHUNK)])

            return carry

        lax.fori_loop(0, max_per_tile, wbody, 0)

    return sc_agg


def _stats_body(nblocks, p0_ref, p1_ref, b_ref, agg_ref, stats_ref, acc_ref):
    i = pl.program_id(0)
    agg = p0_ref[0] + p1_ref[0] + b_ref[...]
    agg_ref[...] = agg

    @pl.when(i == 0)
    def _():
        acc_ref[...] = jnp.zeros_like(acc_ref)

    acc_ref[0, :] += jnp.sum(agg, axis=0)
    acc_ref[1, :] += jnp.sum(agg * agg, axis=0)

    @pl.when(i == nblocks - 1)
    def _():
        stats_ref[...] = acc_ref[...]


def _norm_body(n_rows, agg_ref, stats_ref, gamma_ref, beta_ref, out_ref):
    mean = stats_ref[0:1, :] * (1.0 / n_rows)
    ex2 = stats_ref[1:2, :] * (1.0 / n_rows)
    var = ex2 - mean * mean
    scale = jax.lax.rsqrt(var + EPS) * gamma_ref[...]
    out_ref[...] = (agg_ref[...] - mean) * scale + beta_ref[...]


def kernel(x, edge_index, W, b, gamma, beta):
    n, d = x.shape
    e = edge_index.shape[1]
    block_rows = 1000

    h = _matmul(x, W, block_rows)

    src = edge_index[0]
    dst = edge_index[1]
    zeros = jnp.zeros((CHUNK, d), jnp.float32)
    partial = _make_sc_agg(n, d, e)(h, src, dst, zeros)

    nblocks = n // block_rows
    b2 = b.reshape(1, d)
    agg, stats = pl.pallas_call(
        functools.partial(_stats_body, nblocks),
        grid=(nblocks,),
        in_specs=[
            pl.BlockSpec((1, block_rows, d), lambda i: (0, i, 0)),
            pl.BlockSpec((1, block_rows, d), lambda i: (1, i, 0)),
            pl.BlockSpec((1, d), lambda i: (0, 0)),
        ],
        out_specs=[
            pl.BlockSpec((block_rows, d), lambda i: (i, 0)),
            pl.BlockSpec((8, d), lambda i: (0, 0)),
        ],
        out_shape=[
            jax.ShapeDtypeStruct((n, d), jnp.float32),
            jax.ShapeDtypeStruct((8, d), jnp.float32),
        ],
        scratch_shapes=[pltpu.VMEM((8, d), jnp.float32)],
    )(partial, partial, b2)

    out = pl.pallas_call(
        functools.partial(_norm_body, float(n)),
        grid=(nblocks,),
        in_specs=[
            pl.BlockSpec((block_rows, d), lambda i: (i, 0)),
            pl.BlockSpec((8, d), lambda i: (0, 0)),
            pl.BlockSpec((1, d), lambda i: (0, 0)),
            pl.BlockSpec((1, d), lambda i: (0, 0)),
        ],
        out_specs=pl.BlockSpec((block_rows, d), lambda i: (i, 0)),
        out_shape=jax.ShapeDtypeStruct((n, d), jnp.float32),
    )(agg, stats, gamma.reshape(1, d), beta.reshape(1, d))

    return out


# speedup vs baseline: 8.8264x; 1.7317x over previous
import functools

import jax
import jax.numpy as jnp
from jax import lax
from jax.experimental import pallas as pl
from jax.experimental.pallas import tpu as pltpu
from jax.experimental.pallas import tpu_sc as plsc

EPS = 1e-5

NC = 2
NS = 16
CHUNK = 80


def _matmul_body(x_ref, w_ref, h_ref):
    h_ref[...] = jnp.dot(x_ref[...], w_ref[...],
                         preferred_element_type=jnp.float32)


def _matmul(x, W, block_rows):
    n, d = x.shape
    return pl.pallas_call(
        _matmul_body,
        grid=(n // block_rows,),
        in_specs=[
            pl.BlockSpec((block_rows, d), lambda i: (i, 0)),
            pl.BlockSpec((d, d), lambda i: (0, 0)),
        ],
        out_specs=pl.BlockSpec((block_rows, d), lambda i: (i, 0)),
        out_shape=jax.ShapeDtypeStruct((n, d), jnp.float32),
    )(x, W)


def _make_sc_agg(n, d, e):
    per_w = e // (NC * NS)
    chunks = per_w // CHUNK
    rem = per_w - chunks * CHUNK
    assert rem == 0, "edge count must split evenly into chunks"
    nzch = n // CHUNK
    max_per_tile = (nzch + NS - 1) // NS

    mesh = plsc.VectorSubcoreMesh(core_axis_name="c", subcore_axis_name="s")

    nbuf = 3
    groups = chunks // nbuf
    tail = chunks - groups * nbuf

    @functools.partial(
        pl.kernel,
        mesh=mesh,
        out_type=jax.ShapeDtypeStruct((NC, n, d), jnp.float32),
        compiler_params=pltpu.CompilerParams(use_tc_tiling_on_sc=False),
        scratch_types=[
            pltpu.VMEM((chunks, 1, CHUNK), jnp.int32),
            pltpu.VMEM((chunks, 1, CHUNK), jnp.int32),
            [pltpu.VMEM((CHUNK, d), jnp.float32) for _ in range(nbuf)],
            pltpu.VMEM_SHARED((n, d), jnp.float32),
            [pltpu.SemaphoreType.DMA for _ in range(nbuf)],
            pltpu.SemaphoreType.DMA,
        ],
    )
    def sc_agg(h_hbm, src_hbm, dst_hbm, zero_hbm, out_hbm,
               sidx, didx, rows, acc, gsems, ssem):
        cid = lax.axis_index("c")
        sid = lax.axis_index("s")
        wid = cid * NS + sid

        def zbody(t, carry):
            c = sid + t * NS

            @pl.when(c < nzch)
            def _():
                pltpu.sync_copy(zero_hbm, acc.at[pl.ds(c * CHUNK, CHUNK)])

            return carry

        lax.fori_loop(0, max_per_tile, zbody, 0)

        pltpu.sync_copy(src_hbm.at[wid], sidx)
        pltpu.sync_copy(dst_hbm.at[wid], didx)
        plsc.subcore_barrier()

        def group_body(t, carry):
            c0 = t * nbuf
            descs = []
            for b in range(nbuf):
                descs.append(pltpu.async_copy(
                    h_hbm.at[sidx.at[c0 + b, 0]], rows[b], gsems[b]))
            sdescs = []
            for b in range(nbuf):
                descs[b].wait()
                sdescs.append(pltpu.async_copy(
                    rows[b], acc.at[didx.at[c0 + b, 0]], ssem, add=True))
            for b in range(nbuf):
                sdescs[b].wait()
            return carry

        lax.fori_loop(0, groups, group_body, 0)
        for j in range(tail):
            c = groups * nbuf + j
            pltpu.async_copy(h_hbm.at[sidx.at[c, 0]], rows[j], gsems[j]).wait()
            pltpu.sync_copy(rows[j], acc.at[didx.at[c, 0]], add=True)
        plsc.subcore_barrier()

        def wbody(t, carry):
            c = sid + t * NS

            @pl.when(c < nzch)
            def _():
                pltpu.sync_copy(acc.at[pl.ds(c * CHUNK, CHUNK)],
                                out_hbm.at[cid, pl.ds(c * CHUNK, CHUNK)])

            return carry

        lax.fori_loop(0, max_per_tile, wbody, 0)

    return sc_agg


def _stats_body(nblocks, p0_ref, p1_ref, b_ref, agg_ref, stats_ref, acc_ref):
    i = pl.program_id(0)
    agg = p0_ref[0] + p1_ref[0] + b_ref[...]
    agg_ref[...] = agg

    @pl.when(i == 0)
    def _():
        acc_ref[...] = jnp.zeros_like(acc_ref)

    acc_ref[0, :] += jnp.sum(agg, axis=0)
    acc_ref[1, :] += jnp.sum(agg * agg, axis=0)

    @pl.when(i == nblocks - 1)
    def _():
        stats_ref[...] = acc_ref[...]


def _norm_body(n_rows, agg_ref, stats_ref, gamma_ref, beta_ref, out_ref):
    mean = stats_ref[0:1, :] * (1.0 / n_rows)
    ex2 = stats_ref[1:2, :] * (1.0 / n_rows)
    var = ex2 - mean * mean
    scale = jax.lax.rsqrt(var + EPS) * gamma_ref[...]
    out_ref[...] = (agg_ref[...] - mean) * scale + beta_ref[...]


def kernel(x, edge_index, W, b, gamma, beta):
    n, d = x.shape
    e = edge_index.shape[1]
    block_rows = 1000

    h = _matmul(x, W, block_rows)

    chunks = e // (NC * NS) // CHUNK
    src = edge_index[0].reshape(NC * NS, chunks, 1, CHUNK)
    dst = edge_index[1].reshape(NC * NS, chunks, 1, CHUNK)
    zeros = jnp.zeros((CHUNK, d), jnp.float32)
    partial = _make_sc_agg(n, d, e)(h, src, dst, zeros)

    nblocks = n // block_rows
    b2 = b.reshape(1, d)
    agg, stats = pl.pallas_call(
        functools.partial(_stats_body, nblocks),
        grid=(nblocks,),
        in_specs=[
            pl.BlockSpec((1, block_rows, d), lambda i: (0, i, 0)),
            pl.BlockSpec((1, block_rows, d), lambda i: (1, i, 0)),
            pl.BlockSpec((1, d), lambda i: (0, 0)),
        ],
        out_specs=[
            pl.BlockSpec((block_rows, d), lambda i: (i, 0)),
            pl.BlockSpec((8, d), lambda i: (0, 0)),
        ],
        out_shape=[
            jax.ShapeDtypeStruct((n, d), jnp.float32),
            jax.ShapeDtypeStruct((8, d), jnp.float32),
        ],
        scratch_shapes=[pltpu.VMEM((8, d), jnp.float32)],
    )(partial, partial, b2)

    out = pl.pallas_call(
        functools.partial(_norm_body, float(n)),
        grid=(nblocks,),
        in_specs=[
            pl.BlockSpec((block_rows, d), lambda i: (i, 0)),
            pl.BlockSpec((8, d), lambda i: (0, 0)),
            pl.BlockSpec((1, d), lambda i: (0, 0)),
            pl.BlockSpec((1, d), lambda i: (0, 0)),
        ],
        out_specs=pl.BlockSpec((block_rows, d), lambda i: (i, 0)),
        out_shape=jax.ShapeDtypeStruct((n, d), jnp.float32),
    )(agg, stats, gamma.reshape(1, d), beta.reshape(1, d))

    return out
